# baseline (speedup 1.0000x reference)
import functools

import jax
import jax.numpy as jnp
import numpy as np
from jax.experimental import pallas as pl
from jax.experimental.pallas import tpu as pltpu

_B = 400


def _xw_kernel(x_ref, w_ref, o_ref):
    o_ref[:, :] = jnp.dot(x_ref[:, :], w_ref[:, :],
                          preferred_element_type=jnp.float32)


def _passA_kernel(adj_ref, g_ref, b1_ref, w2_ref, p_ref, opart_ref, pscr_ref):
    i = pl.program_id(0)

    @pl.when(i == 0)
    def _init():
        pscr_ref[:, :] = jnp.zeros_like(pscr_ref)

    a = adj_ref[:, :]
    h = jnp.dot(a, g_ref[:, :], preferred_element_type=jnp.float32)
    h = jnp.maximum(h + b1_ref[:], 0.0)
    pb = jnp.dot(h, w2_ref[:, :], preferred_element_type=jnp.float32)
    pscr_ref[pl.ds(i * _B, _B), :] = pb
    p_ref[:, :] = pb
    opart_ref[:, :] = jnp.dot(a, pscr_ref[:, :],
                              preferred_element_type=jnp.float32)


_W = 512
_N = 10000
_K = (_N + _W - 1) // _W


def _passB_kernel(info_ref, adj_ref, p_ref, opart_ref, b2_ref, o_ref, acc_ref):
    t = pl.program_id(0)
    i = info_ref[0, t]
    k = info_ref[1, t]
    first = info_ref[2, t]
    last = info_ref[3, t]

    @pl.when(first == 1)
    def _load():
        acc_ref[:, :] = opart_ref[:, :]

    rowg = k * _W + jax.lax.broadcasted_iota(jnp.int32, (_W, 1), 0)
    lo = jnp.where(first == 1, (i + 1) * _B, 0)
    keep = (rowg >= lo) & (rowg < _N)
    pm = jnp.where(keep, p_ref[:, :], 0.0)

    @pl.when(k != _K - 1)
    def _acc():
        acc_ref[:, :] += jnp.dot(adj_ref[:, :], pm,
                                 preferred_element_type=jnp.float32)

    @pl.when(k == _K - 1)
    def _acc_edge():
        colg = k * _W + jax.lax.broadcasted_iota(jnp.int32, (_B, _W), 1)
        am = jnp.where(colg < _N, adj_ref[:, :], 0.0)
        acc_ref[:, :] += jnp.dot(am, pm, preferred_element_type=jnp.float32)

    @pl.when(last == 1)
    def _epilogue():
        o = acc_ref[:, :] + b2_ref[:]
        m = jnp.max(o, axis=1, keepdims=True)
        e = o - m
        lse = jnp.log(jnp.sum(jnp.exp(e), axis=1, keepdims=True))
        o_ref[:, :] = e - lse


def _upper_tile_table(T):
    rows = []
    for i in range(T):
        k0 = ((i + 1) * _B) // _W
        for k in range(k0, _K):
            rows.append((i, k, int(k == k0), int(k == _K - 1)))
    return np.asarray(rows, dtype=np.int32).T


@jax.jit
def _run(x, adj, W1, b1, W2, b2):
    N, nfeat = x.shape
    nhid = W1.shape[1]
    nclass = W2.shape[1]
    B = _B
    T = N // B

    g = pl.pallas_call(
        _xw_kernel,
        out_shape=jax.ShapeDtypeStruct((N, nhid), jnp.float32),
    )(x, W1)

    p, opart = pl.pallas_call(
        _passA_kernel,
        grid=(T,),
        in_specs=[
            pl.BlockSpec((B, N), lambda i: (i, 0)),
            pl.BlockSpec((N, nhid), lambda i: (0, 0)),
            pl.BlockSpec((nhid,), lambda i: (0,)),
            pl.BlockSpec((nhid, nclass), lambda i: (0, 0)),
        ],
        out_specs=[
            pl.BlockSpec((B, nclass), lambda i: (i, 0)),
            pl.BlockSpec((B, nclass), lambda i: (i, 0)),
        ],
        out_shape=[
            jax.ShapeDtypeStruct((N, nclass), jnp.float32),
            jax.ShapeDtypeStruct((N, nclass), jnp.float32),
        ],
        scratch_shapes=[pltpu.VMEM((N, nclass), jnp.float32)],
    )(adj, g, b1, W2)

    info = jnp.asarray(_upper_tile_table(T))
    ntiles = info.shape[1]

    grid_spec = pltpu.PrefetchScalarGridSpec(
        num_scalar_prefetch=1,
        grid=(ntiles,),
        in_specs=[
            pl.BlockSpec((B, _W), lambda t, info: (info[0, t], info[1, t])),
            pl.BlockSpec((_W, nclass), lambda t, info: (info[1, t], 0)),
            pl.BlockSpec((B, nclass), lambda t, info: (info[0, t], 0)),
            pl.BlockSpec((nclass,), lambda t, info: (0,)),
        ],
        out_specs=pl.BlockSpec((B, nclass), lambda t, info: (info[0, t], 0)),
        scratch_shapes=[pltpu.VMEM((B, nclass), jnp.float32)],
    )

    out = pl.pallas_call(
        _passB_kernel,
        grid_spec=grid_spec,
        out_shape=jax.ShapeDtypeStruct((N, nclass), jnp.float32),
    )(info, adj, p, opart, b2)

    return out


def kernel(x, adj, W1, b1, W2, b2, epoch, test):
    del epoch, test
    return _run(x, adj, W1, b1, W2, b2)


# speedup vs baseline: 1.5188x; 1.5188x over previous
import functools

import jax
import jax.numpy as jnp
import numpy as np
from jax.experimental import pallas as pl
from jax.experimental.pallas import tpu as pltpu

_N = 10000
_B = 400
_T = _N // _B
_W = 1024
_K = (_N + _W - 1) // _W
_EDGE = _N - (_K - 1) * _W
_NH = 64
_NC = 16


def _xw_kernel(x_ref, w_ref, o_ref):
    o_ref[:, :] = jnp.dot(x_ref[:, :], w_ref[:, :],
                          preferred_element_type=jnp.float32)


def _passA_kernel(adj_ref, g_ref, b1_ref, w2_ref, p_ref, opart_ref, gp_ref):
    i = pl.program_id(0)

    @pl.when(i == 0)
    def _init():
        gp_ref[:, 0:_NH] = g_ref[:, :]
        gp_ref[:, _NH:_NH + _NC] = jnp.zeros((_N, _NC), jnp.float32)

    a = adj_ref[:, :]
    hp = jnp.dot(a, gp_ref[:, :], preferred_element_type=jnp.float32)
    h = jnp.maximum(hp[:, 0:_NH] + b1_ref[:], 0.0)
    pb = jnp.dot(h, w2_ref[:, :], preferred_element_type=jnp.float32)
    gp_ref[pl.ds(i * _B, _B), _NH:_NH + _NC] = pb
    p_ref[:, :] = pb
    opart_ref[:, :] = hp[:, _NH:_NH + _NC]


def _passB_kernel(info_ref, adj_ref, p_ref, opart_ref, b2_ref, o_ref, acc_ref):
    t = pl.program_id(0)
    i = info_ref[0, t]
    k = info_ref[1, t]
    first = info_ref[2, t]
    last = info_ref[3, t]

    @pl.when(first == 1)
    def _load():
        acc_ref[:, :] = opart_ref[:, :]

    rowg = k * _W + jax.lax.broadcasted_iota(jnp.int32, (_W, 1), 0)
    lo = jnp.where(first == 1, i * _B, 0)
    pm = jnp.where(rowg >= lo, p_ref[:, :], 0.0)

    @pl.when(k != _K - 1)
    def _acc():
        acc_ref[:, :] += jnp.dot(adj_ref[:, :], pm,
                                 preferred_element_type=jnp.float32)

    @pl.when(k == _K - 1)
    def _acc_edge():
        acc_ref[:, :] += jnp.dot(adj_ref[:, 0:_EDGE], pm[0:_EDGE, :],
                                 preferred_element_type=jnp.float32)

    @pl.when(last == 1)
    def _epilogue():
        o = acc_ref[:, :] + b2_ref[:]
        m = jnp.max(o, axis=1, keepdims=True)
        e = o - m
        lse = jnp.log(jnp.sum(jnp.exp(e), axis=1, keepdims=True))
        o_ref[:, :] = e - lse


def _tile_table():
    rows = []
    for i in range(_T):
        k0 = (i * _B) // _W
        for k in range(k0, _K):
            rows.append((i, k, int(k == k0), int(k == _K - 1)))
    return np.asarray(rows, dtype=np.int32).T


@jax.jit
def _run(x, adj, W1, b1, W2, b2):
    N, nfeat = x.shape
    nhid = W1.shape[1]
    nclass = W2.shape[1]

    g = pl.pallas_call(
        _xw_kernel,
        out_shape=jax.ShapeDtypeStruct((N, nhid), jnp.float32),
    )(x, W1)

    p, opart = pl.pallas_call(
        _passA_kernel,
        grid=(_T,),
        in_specs=[
            pl.BlockSpec((_B, N), lambda i: (i, 0)),
            pl.BlockSpec((N, nhid), lambda i: (0, 0)),
            pl.BlockSpec((nhid,), lambda i: (0,)),
            pl.BlockSpec((nhid, nclass), lambda i: (0, 0)),
        ],
        out_specs=[
            pl.BlockSpec((_B, nclass), lambda i: (i, 0)),
            pl.BlockSpec((_B, nclass), lambda i: (i, 0)),
        ],
        out_shape=[
            jax.ShapeDtypeStruct((N, nclass), jnp.float32),
            jax.ShapeDtypeStruct((N, nclass), jnp.float32),
        ],
        scratch_shapes=[pltpu.VMEM((N, _NH + _NC), jnp.float32)],
    )(adj, g, b1, W2)

    info = jnp.asarray(_tile_table())
    ntiles = info.shape[1]

    grid_spec = pltpu.PrefetchScalarGridSpec(
        num_scalar_prefetch=1,
        grid=(ntiles,),
        in_specs=[
            pl.BlockSpec((_B, _W), lambda t, info: (info[0, t], info[1, t])),
            pl.BlockSpec((_W, nclass), lambda t, info: (info[1, t], 0)),
            pl.BlockSpec((_B, nclass), lambda t, info: (info[0, t], 0)),
            pl.BlockSpec((nclass,), lambda t, info: (0,)),
        ],
        out_specs=pl.BlockSpec((_B, nclass), lambda t, info: (info[0, t], 0)),
        scratch_shapes=[pltpu.VMEM((_B, nclass), jnp.float32)],
    )

    out = pl.pallas_call(
        _passB_kernel,
        grid_spec=grid_spec,
        out_shape=jax.ShapeDtypeStruct((N, nclass), jnp.float32),
    )(info, adj, p, opart, b2)

    return out


def kernel(x, adj, W1, b1, W2, b2, epoch, test):
    del epoch, test
    return _run(x, adj, W1, b1, W2, b2)


# speedup vs baseline: 1.5227x; 1.0026x over previous
import functools

import jax
import jax.numpy as jnp
import numpy as np
from jax.experimental import pallas as pl
from jax.experimental.pallas import tpu as pltpu

_N = 10000
_B = 400
_T = _N // _B
_W = 1024
_K = (_N + _W - 1) // _W
_EDGE = _N - (_K - 1) * _W
_NH = 64
_NC = 16


def _xw_kernel(x_ref, w_ref, o_ref):
    o_ref[:, :] = jnp.dot(x_ref[:, :], w_ref[:, :],
                          preferred_element_type=jnp.float32)


def _passA_kernel(adj_ref, g_ref, b1_ref, w2_ref, p_ref, opart_ref, gp_ref):
    i = pl.program_id(0)

    @pl.when(i == 0)
    def _init():
        gp_ref[:, 0:_NH] = g_ref[:, :]
        gp_ref[:, _NH:_NH + _NC] = jnp.zeros((_N, _NC), jnp.float32)

    a = adj_ref[:, :]
    hp = jnp.dot(a, gp_ref[:, :], preferred_element_type=jnp.float32)
    h = jnp.maximum(hp[:, 0:_NH] + b1_ref[:], 0.0)
    pb = jnp.dot(h, w2_ref[:, :], preferred_element_type=jnp.float32)
    gp_ref[pl.ds(i * _B, _B), _NH:_NH + _NC] = pb
    p_ref[:, :] = pb
    opart_ref[:, :] = hp[:, _NH:_NH + _NC]


def _passB_kernel(info_ref, adj_ref, p_ref, opart_ref, b2_ref, o_ref, acc_ref):
    t = pl.program_id(0)
    i = info_ref[0, t]
    k = info_ref[1, t]
    first = info_ref[2, t]
    last = info_ref[3, t]

    @pl.when(first == 1)
    def _load():
        acc_ref[:, :] = opart_ref[:, :]

    rowg = k * _W + jax.lax.broadcasted_iota(jnp.int32, (_W, 1), 0)
    lo = jnp.where(first == 1, i * _B, 0)
    pm = jnp.where(rowg >= lo, p_ref[:, :], 0.0).astype(jnp.bfloat16)

    @pl.when(k != _K - 1)
    def _acc():
        acc_ref[:, :] += jnp.dot(adj_ref[:, :].astype(jnp.bfloat16), pm,
                                 preferred_element_type=jnp.float32)

    @pl.when(k == _K - 1)
    def _acc_edge():
        acc_ref[:, :] += jnp.dot(adj_ref[:, 0:_EDGE].astype(jnp.bfloat16),
                                 pm[0:_EDGE, :],
                                 preferred_element_type=jnp.float32)

    @pl.when(last == 1)
    def _epilogue():
        o = acc_ref[:, :] + b2_ref[:]
        m = jnp.max(o, axis=1, keepdims=True)
        e = o - m
        lse = jnp.log(jnp.sum(jnp.exp(e), axis=1, keepdims=True))
        o_ref[:, :] = e - lse


def _tile_table():
    rows = []
    for i in range(_T):
        k0 = (i * _B) // _W
        for k in range(k0, _K):
            rows.append((i, k, int(k == k0), int(k == _K - 1)))
    return np.asarray(rows, dtype=np.int32).T


@jax.jit
def _run(x, adj, W1, b1, W2, b2):
    N, nfeat = x.shape
    nhid = W1.shape[1]
    nclass = W2.shape[1]

    g = pl.pallas_call(
        _xw_kernel,
        out_shape=jax.ShapeDtypeStruct((N, nhid), jnp.float32),
    )(x, W1)

    p, opart = pl.pallas_call(
        _passA_kernel,
        grid=(_T,),
        in_specs=[
            pl.BlockSpec((_B, N), lambda i: (i, 0)),
            pl.BlockSpec((N, nhid), lambda i: (0, 0)),
            pl.BlockSpec((nhid,), lambda i: (0,)),
            pl.BlockSpec((nhid, nclass), lambda i: (0, 0)),
        ],
        out_specs=[
            pl.BlockSpec((_B, nclass), lambda i: (i, 0)),
            pl.BlockSpec((_B, nclass), lambda i: (i, 0)),
        ],
        out_shape=[
            jax.ShapeDtypeStruct((N, nclass), jnp.float32),
            jax.ShapeDtypeStruct((N, nclass), jnp.float32),
        ],
        scratch_shapes=[pltpu.VMEM((N, _NH + _NC), jnp.float32)],
    )(adj, g, b1, W2)

    info = jnp.asarray(_tile_table())
    ntiles = info.shape[1]

    grid_spec = pltpu.PrefetchScalarGridSpec(
        num_scalar_prefetch=1,
        grid=(ntiles,),
        in_specs=[
            pl.BlockSpec((_B, _W), lambda t, info: (info[0, t], info[1, t])),
            pl.BlockSpec((_W, nclass), lambda t, info: (info[1, t], 0)),
            pl.BlockSpec((_B, nclass), lambda t, info: (info[0, t], 0)),
            pl.BlockSpec((nclass,), lambda t, info: (0,)),
        ],
        out_specs=pl.BlockSpec((_B, nclass), lambda t, info: (info[0, t], 0)),
        scratch_shapes=[pltpu.VMEM((_B, nclass), jnp.float32)],
    )

    out = pl.pallas_call(
        _passB_kernel,
        grid_spec=grid_spec,
        out_shape=jax.ShapeDtypeStruct((N, nclass), jnp.float32),
    )(info, adj, p, opart, b2)

    return out


def kernel(x, adj, W1, b1, W2, b2, epoch, test):
    del epoch, test
    return _run(x, adj, W1, b1, W2, b2)


# speedup vs baseline: 1.7120x; 1.1243x over previous
import functools

import jax
import jax.numpy as jnp
import numpy as np
from jax.experimental import pallas as pl
from jax.experimental.pallas import tpu as pltpu

_N = 10000
_B = 400
_T = _N // _B
_W = 2048
_K = (_N + _W - 1) // _W
_EDGE = _N - (_K - 1) * _W
_NH = 64
_NC = 16


def _xw_kernel(x_ref, w_ref, o_ref):
    o_ref[:, :] = jnp.dot(x_ref[:, :], w_ref[:, :],
                          preferred_element_type=jnp.float32)


def _passA_kernel(adj_ref, g_ref, b1_ref, w2_ref, p_ref, opart_ref, gp_ref):
    i = pl.program_id(0)

    @pl.when(i == 0)
    def _init():
        gp_ref[:, 0:_NH] = g_ref[:, :]
        gp_ref[:, _NH:_NH + _NC] = jnp.zeros((_N, _NC), jnp.float32)

    a = adj_ref[:, :]
    hp = jnp.dot(a, gp_ref[:, :], preferred_element_type=jnp.float32)
    h = jnp.maximum(hp[:, 0:_NH] + b1_ref[:], 0.0)
    pb = jnp.dot(h, w2_ref[:, :], preferred_element_type=jnp.float32)
    gp_ref[pl.ds(i * _B, _B), _NH:_NH + _NC] = pb
    p_ref[:, :] = pb
    opart_ref[:, :] = hp[:, _NH:_NH + _NC]


def _passB_kernel(info_ref, adj_ref, p_ref, opart_ref, b2_ref, o_ref, acc_ref):
    t = pl.program_id(0)
    i = info_ref[0, t]
    k = info_ref[1, t]
    first = info_ref[2, t]
    last = info_ref[3, t]

    @pl.when(first == 1)
    def _load():
        acc_ref[:, :] = opart_ref[:, :]

    rowg = k * _W + jax.lax.broadcasted_iota(jnp.int32, (_W, 1), 0)
    lo = jnp.where(first == 1, i * _B, 0)
    pm = jnp.where(rowg >= lo, p_ref[:, :], 0.0).astype(jnp.bfloat16)

    @pl.when(k != _K - 1)
    def _acc():
        acc_ref[:, :] += jnp.dot(adj_ref[:, :].astype(jnp.bfloat16), pm,
                                 preferred_element_type=jnp.float32)

    @pl.when(k == _K - 1)
    def _acc_edge():
        acc_ref[:, :] += jnp.dot(adj_ref[:, 0:_EDGE].astype(jnp.bfloat16),
                                 pm[0:_EDGE, :],
                                 preferred_element_type=jnp.float32)

    @pl.when(last == 1)
    def _epilogue():
        o = acc_ref[:, :] + b2_ref[:]
        m = jnp.max(o, axis=1, keepdims=True)
        e = o - m
        lse = jnp.log(jnp.sum(jnp.exp(e), axis=1, keepdims=True))
        o_ref[:, :] = e - lse


def _tile_table():
    rows = []
    for i in range(_T):
        k0 = (i * _B) // _W
        for k in range(k0, _K):
            rows.append((i, k, int(k == k0), int(k == _K - 1)))
    return np.asarray(rows, dtype=np.int32).T


@jax.jit
def _run(x, adj, W1, b1, W2, b2):
    N, nfeat = x.shape
    nhid = W1.shape[1]
    nclass = W2.shape[1]

    g = pl.pallas_call(
        _xw_kernel,
        out_shape=jax.ShapeDtypeStruct((N, nhid), jnp.float32),
    )(x, W1)

    p, opart = pl.pallas_call(
        _passA_kernel,
        grid=(_T,),
        in_specs=[
            pl.BlockSpec((_B, N), lambda i: (i, 0)),
            pl.BlockSpec((N, nhid), lambda i: (0, 0)),
            pl.BlockSpec((nhid,), lambda i: (0,)),
            pl.BlockSpec((nhid, nclass), lambda i: (0, 0)),
        ],
        out_specs=[
            pl.BlockSpec((_B, nclass), lambda i: (i, 0)),
            pl.BlockSpec((_B, nclass), lambda i: (i, 0)),
        ],
        out_shape=[
            jax.ShapeDtypeStruct((N, nclass), jnp.float32),
            jax.ShapeDtypeStruct((N, nclass), jnp.float32),
        ],
        scratch_shapes=[pltpu.VMEM((N, _NH + _NC), jnp.float32)],
    )(adj, g, b1, W2)

    info = jnp.asarray(_tile_table())
    ntiles = info.shape[1]

    grid_spec = pltpu.PrefetchScalarGridSpec(
        num_scalar_prefetch=1,
        grid=(ntiles,),
        in_specs=[
            pl.BlockSpec((_B, _W), lambda t, info: (info[0, t], info[1, t])),
            pl.BlockSpec((_W, nclass), lambda t, info: (info[1, t], 0)),
            pl.BlockSpec((_B, nclass), lambda t, info: (info[0, t], 0)),
            pl.BlockSpec((nclass,), lambda t, info: (0,)),
        ],
        out_specs=pl.BlockSpec((_B, nclass), lambda t, info: (info[0, t], 0)),
        scratch_shapes=[pltpu.VMEM((_B, nclass), jnp.float32)],
    )

    out = pl.pallas_call(
        _passB_kernel,
        grid_spec=grid_spec,
        out_shape=jax.ShapeDtypeStruct((N, nclass), jnp.float32),
    )(info, adj, p, opart, b2)

    return out


def kernel(x, adj, W1, b1, W2, b2, epoch, test):
    del epoch, test
    return _run(x, adj, W1, b1, W2, b2)


# speedup vs baseline: 1.8724x; 1.0937x over previous
import functools

import jax
import jax.numpy as jnp
import numpy as np
from jax.experimental import pallas as pl
from jax.experimental.pallas import tpu as pltpu

_N = 10000
_B = 512
_T = (_N + _B - 1) // _B
_W = 2048
_K = (_N + _W - 1) // _W
_EDGE = _N - (_K - 1) * _W
_CHUNK = _W // _B
_NH = 64
_NC = 16
_NPAD = _T * _B


def _xw_kernel(x_ref, w_ref, o_ref):
    o_ref[:, :] = jnp.dot(x_ref[:, :], w_ref[:, :],
                          preferred_element_type=jnp.float32)


def _passA_kernel(adj_ref, g_ref, b1_ref, w2_ref, p_ref, opart_ref,
                  gp_ref, stage_ref):
    i = pl.program_id(0)

    @pl.when(i == 0)
    def _init():
        gp_ref[:, :] = jnp.zeros((_NPAD, _NH + _NC), jnp.float32)
        gp_ref[0:_N, 0:_NH] = g_ref[:, :]

    @pl.when((i % _CHUNK == 0) & (i > 0))
    def _expose():
        c = i // _CHUNK - 1
        gp_ref[pl.ds(c * _W, _W), _NH:_NH + _NC] = (
            stage_ref[pl.ds(c * _W, _W), :])

    a = adj_ref[:, :]
    hp = jnp.dot(a, gp_ref[0:_N, :], preferred_element_type=jnp.float32)
    h = jnp.maximum(hp[:, 0:_NH] + b1_ref[:], 0.0)
    pb = jnp.dot(h, w2_ref[:, :], preferred_element_type=jnp.float32)
    stage_ref[pl.ds(i * _B, _B), :] = pb
    p_ref[:, :] = pb
    opart_ref[:, :] = hp[:, _NH:_NH + _NC]


def _passB_kernel(info_ref, adj_ref, p_ref, opart_ref, b2_ref, o_ref, acc_ref):
    t = pl.program_id(0)
    k = info_ref[1, t]
    first = info_ref[2, t]
    last = info_ref[3, t]

    @pl.when(first == 1)
    def _load():
        acc_ref[:, :] = opart_ref[:, :]

    @pl.when(k != _K - 1)
    def _acc():
        acc_ref[:, :] += jnp.dot(adj_ref[:, :], p_ref[:, :],
                                 preferred_element_type=jnp.float32,
                                 precision=jax.lax.Precision.DEFAULT)

    @pl.when(k == _K - 1)
    def _acc_edge():
        acc_ref[:, :] += jnp.dot(adj_ref[:, 0:_EDGE], p_ref[0:_EDGE, :],
                                 preferred_element_type=jnp.float32,
                                 precision=jax.lax.Precision.DEFAULT)

    @pl.when(last == 1)
    def _epilogue():
        o = acc_ref[:, :] + b2_ref[:]
        m = jnp.max(o, axis=1, keepdims=True)
        e = o - m
        lse = jnp.log(jnp.sum(jnp.exp(e), axis=1, keepdims=True))
        o_ref[:, :] = e - lse


def _tile_table():
    rows = []
    for i in range(_T):
        k0 = i // _CHUNK
        for k in range(k0, _K):
            rows.append((i, k, int(k == k0), int(k == _K - 1)))
    return np.asarray(rows, dtype=np.int32).T


@jax.jit
def _run(x, adj, W1, b1, W2, b2):
    N, nfeat = x.shape
    nhid = W1.shape[1]
    nclass = W2.shape[1]

    g = pl.pallas_call(
        _xw_kernel,
        out_shape=jax.ShapeDtypeStruct((N, nhid), jnp.float32),
    )(x, W1)

    p, opart = pl.pallas_call(
        _passA_kernel,
        grid=(_T,),
        in_specs=[
            pl.BlockSpec((_B, N), lambda i: (i, 0)),
            pl.BlockSpec((N, nhid), lambda i: (0, 0)),
            pl.BlockSpec((nhid,), lambda i: (0,)),
            pl.BlockSpec((nhid, nclass), lambda i: (0, 0)),
        ],
        out_specs=[
            pl.BlockSpec((_B, nclass), lambda i: (i, 0)),
            pl.BlockSpec((_B, nclass), lambda i: (i, 0)),
        ],
        out_shape=[
            jax.ShapeDtypeStruct((N, nclass), jnp.float32),
            jax.ShapeDtypeStruct((N, nclass), jnp.float32),
        ],
        scratch_shapes=[
            pltpu.VMEM((_NPAD, _NH + _NC), jnp.float32),
            pltpu.VMEM((_NPAD, _NC), jnp.float32),
        ],
    )(adj, g, b1, W2)

    info = jnp.asarray(_tile_table())
    ntiles = info.shape[1]

    grid_spec = pltpu.PrefetchScalarGridSpec(
        num_scalar_prefetch=1,
        grid=(ntiles,),
        in_specs=[
            pl.BlockSpec((_B, _W), lambda t, info: (info[0, t], info[1, t])),
            pl.BlockSpec((_W, nclass), lambda t, info: (info[1, t], 0)),
            pl.BlockSpec((_B, nclass), lambda t, info: (info[0, t], 0)),
            pl.BlockSpec((nclass,), lambda t, info: (0,)),
        ],
        out_specs=pl.BlockSpec((_B, nclass), lambda t, info: (info[0, t], 0)),
        scratch_shapes=[pltpu.VMEM((_B, nclass), jnp.float32)],
    )

    out = pl.pallas_call(
        _passB_kernel,
        grid_spec=grid_spec,
        out_shape=jax.ShapeDtypeStruct((N, nclass), jnp.float32),
    )(info, adj, p, opart, b2)

    return out


def kernel(x, adj, W1, b1, W2, b2, epoch, test):
    del epoch, test
    return _run(x, adj, W1, b1, W2, b2)


# speedup vs baseline: 2.0535x; 1.0967x over previous
import functools

import jax
import jax.numpy as jnp
import numpy as np
from jax.experimental import pallas as pl
from jax.experimental.pallas import tpu as pltpu

_N = 10000
_B = 512
_T = (_N + _B - 1) // _B
_W = 2048
_K = (_N + _W - 1) // _W
_EDGE = _N - (_K - 1) * _W
_CHUNK = _W // _B
_NH = 64
_NC = 16
_NPAD = _T * _B


def _xw_kernel(x_ref, w_ref, o_ref):
    o_ref[:, :] = jnp.dot(x_ref[:, :], w_ref[:, :],
                          preferred_element_type=jnp.float32)


def _passA_kernel(adj_ref, g_ref, b1_ref, w2_ref, p_ref, opart_ref,
                  gp_ref, stage_ref):
    i = pl.program_id(0)

    @pl.when(i == 0)
    def _init():
        gp_ref[:, :] = jnp.zeros((_NPAD, _NH + _NC), jnp.float32)
        gp_ref[0:_N, 0:_NH] = g_ref[:, :]

    @pl.when((i % _CHUNK == 0) & (i > 0))
    def _expose():
        c = i // _CHUNK - 1
        gp_ref[pl.ds(c * _W, _W), _NH:_NH + _NC] = (
            stage_ref[pl.ds(c * _W, _W), :])

    a = adj_ref[:, :]
    hp = jnp.dot(a, gp_ref[0:_N, :], preferred_element_type=jnp.float32)
    h = jnp.maximum(hp[:, 0:_NH] + b1_ref[:], 0.0)
    pb = jnp.dot(h, w2_ref[:, :], preferred_element_type=jnp.float32)
    stage_ref[pl.ds(i * _B, _B), :] = pb
    p_ref[:, :] = pb
    opart_ref[:, :] = hp[:, _NH:_NH + _NC]


def _passB_kernel(info_ref, adj_ref, p_ref, opart_ref, b2_ref, o_ref, acc_ref):
    t = pl.program_id(0)
    k = info_ref[1, t]
    first = info_ref[2, t]
    last = info_ref[3, t]

    @pl.when(first == 1)
    def _load():
        acc_ref[:, :] = opart_ref[:, :]

    @pl.when(k != _K - 1)
    def _acc():
        acc_ref[:, :] += jnp.dot(adj_ref[:, :], p_ref[:, :],
                                 preferred_element_type=jnp.float32,
                                 precision=jax.lax.Precision.DEFAULT)

    @pl.when(k == _K - 1)
    def _acc_edge():
        acc_ref[:, :] += jnp.dot(adj_ref[:, 0:_EDGE], p_ref[0:_EDGE, :],
                                 preferred_element_type=jnp.float32,
                                 precision=jax.lax.Precision.DEFAULT)

    @pl.when(last == 1)
    def _epilogue():
        o = acc_ref[:, :] + b2_ref[:]
        m = jnp.max(o, axis=1, keepdims=True)
        e = o - m
        lse = jnp.log(jnp.sum(jnp.exp(e), axis=1, keepdims=True))
        o_ref[:, :] = e - lse


_BB = 1024
_TB = _NPAD // _BB


def _tile_table():
    rows = []
    for j in range(_TB):
        k0 = (2 * j) // _CHUNK
        for k in range(k0, _K):
            rows.append((j, k, int(k == k0), int(k == _K - 1)))
    return np.asarray(rows, dtype=np.int32).T


@jax.jit
def _run(x, adj, W1, b1, W2, b2):
    N, nfeat = x.shape
    nhid = W1.shape[1]
    nclass = W2.shape[1]

    g = pl.pallas_call(
        _xw_kernel,
        out_shape=jax.ShapeDtypeStruct((N, nhid), jnp.float32),
    )(x, W1)

    p, opart = pl.pallas_call(
        _passA_kernel,
        grid=(_T,),
        in_specs=[
            pl.BlockSpec((_B, N), lambda i: (i, 0)),
            pl.BlockSpec((N, nhid), lambda i: (0, 0)),
            pl.BlockSpec((nhid,), lambda i: (0,)),
            pl.BlockSpec((nhid, nclass), lambda i: (0, 0)),
        ],
        out_specs=[
            pl.BlockSpec((_B, nclass), lambda i: (i, 0)),
            pl.BlockSpec((_B, nclass), lambda i: (i, 0)),
        ],
        out_shape=[
            jax.ShapeDtypeStruct((N, nclass), jnp.float32),
            jax.ShapeDtypeStruct((N, nclass), jnp.float32),
        ],
        scratch_shapes=[
            pltpu.VMEM((_NPAD, _NH + _NC), jnp.float32),
            pltpu.VMEM((_NPAD, _NC), jnp.float32),
        ],
    )(adj, g, b1, W2)

    info = jnp.asarray(_tile_table())
    ntiles = info.shape[1]

    grid_spec = pltpu.PrefetchScalarGridSpec(
        num_scalar_prefetch=1,
        grid=(ntiles,),
        in_specs=[
            pl.BlockSpec((_BB, _W), lambda t, info: (info[0, t], info[1, t])),
            pl.BlockSpec((_W, nclass), lambda t, info: (info[1, t], 0)),
            pl.BlockSpec((_BB, nclass), lambda t, info: (info[0, t], 0)),
            pl.BlockSpec((nclass,), lambda t, info: (0,)),
        ],
        out_specs=pl.BlockSpec((_BB, nclass), lambda t, info: (info[0, t], 0)),
        scratch_shapes=[pltpu.VMEM((_BB, nclass), jnp.float32)],
    )

    out = pl.pallas_call(
        _passB_kernel,
        grid_spec=grid_spec,
        out_shape=jax.ShapeDtypeStruct((N, nclass), jnp.float32),
    )(info, adj, p, opart, b2)

    return out


def kernel(x, adj, W1, b1, W2, b2, epoch, test):
    del epoch, test
    return _run(x, adj, W1, b1, W2, b2)


# speedup vs baseline: 2.1079x; 1.0265x over previous
import jax
import jax.numpy as jnp
import numpy as np
from jax.experimental import pallas as pl
from jax.experimental.pallas import tpu as pltpu

_N = 10000
_BB = 1024
_TB = (_N + _BB - 1) // _BB
_W = 2048
_K = (_N + _W - 1) // _W
_EDGE = _N - (_K - 1) * _W
_NH = 64
_NC = 16
_NF = _NH + _NC
_NPAD = _TB * _BB


def _xw_kernel(x_ref, w_ref, o_ref):
    o_ref[:, :] = jnp.dot(x_ref[:, :], w_ref[:, :],
                          preferred_element_type=jnp.float32)


def _main_kernel(info_ref, adj_ref, g_ref, b1_ref, w2_ref, b2_ref, o_ref,
                 gp_ref, stage_ref, opart_ref, hacc_ref, acc_ref):
    t = pl.program_id(0)
    j = info_ref[0, t]
    k = info_ref[1, t]
    phase = info_ref[2, t]
    first = info_ref[3, t]
    last = info_ref[4, t]

    @pl.when(t == 0)
    def _init():
        gp_ref[:, :] = jnp.zeros((_NPAD, _NF), jnp.float32)
        gp_ref[0:_N, 0:_NH] = g_ref[:, :]

    @pl.when((phase == 0) & (k != _K - 1))
    def _a_main():
        d = jnp.dot(adj_ref[:, :], gp_ref[pl.ds(k * _W, _W), :],
                    preferred_element_type=jnp.float32)
        @pl.when(first == 1)
        def _set():
            hacc_ref[:, :] = d
        @pl.when(first == 0)
        def _add():
            hacc_ref[:, :] += d

    @pl.when((phase == 0) & (k == _K - 1))
    def _a_edge():
        d = jnp.dot(adj_ref[:, 0:_EDGE], gp_ref[pl.ds(k * _W, _EDGE), :],
                    preferred_element_type=jnp.float32)
        hacc_ref[:, :] += d

    @pl.when((phase == 0) & (last == 1))
    def _a_epilogue():
        h = jnp.maximum(hacc_ref[:, 0:_NH] + b1_ref[:], 0.0)
        pb = jnp.dot(h, w2_ref[:, :], preferred_element_type=jnp.float32)
        stage_ref[pl.ds(j * _BB, _BB), :] = pb
        opart_ref[pl.ds(j * _BB, _BB), :] = hacc_ref[:, _NH:_NF]

        @pl.when(j == _TB - 1)
        def _zero_tail():
            stage_ref[_N:_NPAD, :] = jnp.zeros((_NPAD - _N, _NC), jnp.float32)

        @pl.when(j % 2 == 1)
        def _expose():
            c = j // 2
            gp_ref[pl.ds(c * _W, _W), _NH:_NF] = (
                stage_ref[pl.ds(c * _W, _W), :])

    @pl.when((phase == 1) & (first == 1))
    def _b_load():
        acc_ref[:, :] = opart_ref[pl.ds(j * _BB, _BB), :]

    @pl.when((phase == 1) & (k != _K - 1))
    def _b_main():
        acc_ref[:, :] += jnp.dot(adj_ref[:, :],
                                 gp_ref[pl.ds(k * _W, _W), _NH:_NF],
                                 preferred_element_type=jnp.float32)

    @pl.when((phase == 1) & (k == _K - 1))
    def _b_edge():
        acc_ref[:, :] += jnp.dot(adj_ref[:, 0:_EDGE],
                                 gp_ref[pl.ds(k * _W, _EDGE), _NH:_NF],
                                 preferred_element_type=jnp.float32)

    @pl.when((phase == 1) & (last == 1))
    def _b_epilogue():
        o = acc_ref[:, :] + b2_ref[:]
        m = jnp.max(o, axis=1, keepdims=True)
        e = o - m
        lse = jnp.log(jnp.sum(jnp.exp(e), axis=1, keepdims=True))
        o_ref[:, :] = e - lse


def _schedule():
    rows = []
    for j in range(_TB):
        for k in range(_K):
            rows.append((j, k, 0, int(k == 0), int(k == _K - 1)))
    for j in range(_TB):
        k0 = j // 2
        for k in range(k0, _K):
            rows.append((j, k, 1, int(k == k0), int(k == _K - 1)))
    return np.asarray(rows, dtype=np.int32).T


@jax.jit
def _run(x, adj, W1, b1, W2, b2):
    N, nfeat = x.shape
    nhid = W1.shape[1]
    nclass = W2.shape[1]

    g = pl.pallas_call(
        _xw_kernel,
        out_shape=jax.ShapeDtypeStruct((N, nhid), jnp.float32),
    )(x, W1)

    info = jnp.asarray(_schedule())
    nsteps = info.shape[1]

    grid_spec = pltpu.PrefetchScalarGridSpec(
        num_scalar_prefetch=1,
        grid=(nsteps,),
        in_specs=[
            pl.BlockSpec((_BB, _W), lambda t, info: (info[0, t], info[1, t])),
            pl.BlockSpec((N, nhid), lambda t, info: (0, 0)),
            pl.BlockSpec((nhid,), lambda t, info: (0,)),
            pl.BlockSpec((nhid, nclass), lambda t, info: (0, 0)),
            pl.BlockSpec((nclass,), lambda t, info: (0,)),
        ],
        out_specs=pl.BlockSpec((_BB, nclass), lambda t, info: (info[0, t], 0)),
        scratch_shapes=[
            pltpu.VMEM((_NPAD, _NF), jnp.float32),
            pltpu.VMEM((_NPAD, _NC), jnp.float32),
            pltpu.VMEM((_NPAD, _NC), jnp.float32),
            pltpu.VMEM((_BB, _NF), jnp.float32),
            pltpu.VMEM((_BB, _NC), jnp.float32),
        ],
    )

    out = pl.pallas_call(
        _main_kernel,
        grid_spec=grid_spec,
        out_shape=jax.ShapeDtypeStruct((N, nclass), jnp.float32),
    )(info, adj, g, b1, W2, b2)

    return out


def kernel(x, adj, W1, b1, W2, b2, epoch, test):
    del epoch, test
    return _run(x, adj, W1, b1, W2, b2)


# speedup vs baseline: 2.1317x; 1.0113x over previous
import jax
import jax.numpy as jnp
import numpy as np
from jax.experimental import pallas as pl
from jax.experimental.pallas import tpu as pltpu

_N = 10000
_BA = 256
_TA = (_N + _BA - 1) // _BA
_BB = 1024
_TB = (_N + _BB - 1) // _BB
_W = 2048
_K = (_N + _W - 1) // _W
_EDGE = _N - (_K - 1) * _W
_CHUNK = _W // _BA
_NH = 64
_NC = 16
_NF = _NH + _NC
_NPAD = _TA * _BA


def _xw_kernel(x_ref, w_ref, o_ref):
    o_ref[:, :] = jnp.dot(x_ref[:, :], w_ref[:, :],
                          preferred_element_type=jnp.float32)


def _main_kernel(info_ref, adja_ref, adjb_ref, g_ref, b1_ref, w2_ref, b2_ref,
                 o_ref, gp_ref, stage_ref, opart_ref, acc_ref):
    t = pl.program_id(0)
    i = info_ref[0, t]
    k = info_ref[1, t]
    phase = info_ref[2, t]
    first = info_ref[3, t]
    last = info_ref[4, t]

    @pl.when(t == 0)
    def _init():
        gp_ref[:, :] = jnp.zeros((_NPAD, _NF), jnp.float32)
        gp_ref[0:_N, 0:_NH] = g_ref[:, :]

    @pl.when(phase == 0)
    def _a_step():
        @pl.when((i % _CHUNK == 0) & (i > 0))
        def _expose():
            c = i // _CHUNK - 1
            gp_ref[pl.ds(c * _W, _W), _NH:_NF] = (
                stage_ref[pl.ds(c * _W, _W), :])

        hp = jnp.dot(adja_ref[:, :], gp_ref[0:_N, :],
                     preferred_element_type=jnp.float32)
        h = jnp.maximum(hp[:, 0:_NH] + b1_ref[:], 0.0)
        pb = jnp.dot(h, w2_ref[:, :], preferred_element_type=jnp.float32)
        stage_ref[pl.ds(i * _BA, _BA), :] = pb
        opart_ref[pl.ds(i * _BA, _BA), :] = hp[:, _NH:_NF]

        @pl.when(i == _TA - 1)
        def _finish_p():
            stage_ref[_N:_NPAD, :] = jnp.zeros((_NPAD - _N, _NC), jnp.float32)
            gp_ref[pl.ds((_K - 1) * _W, _W), _NH:_NF] = (
                stage_ref[pl.ds((_K - 1) * _W, _W), :])

    @pl.when((phase == 1) & (first == 1))
    def _b_load():
        acc_ref[:, :] = opart_ref[pl.ds(i * _BB, _BB), :]

    @pl.when((phase == 1) & (k != _K - 1))
    def _b_main():
        acc_ref[:, :] += jnp.dot(adjb_ref[:, :],
                                 gp_ref[pl.ds(k * _W, _W), _NH:_NF],
                                 preferred_element_type=jnp.float32)

    @pl.when((phase == 1) & (k == _K - 1))
    def _b_edge():
        acc_ref[:, :] += jnp.dot(adjb_ref[:, 0:_EDGE],
                                 gp_ref[pl.ds(k * _W, _EDGE), _NH:_NF],
                                 preferred_element_type=jnp.float32)

    @pl.when((phase == 1) & (last == 1))
    def _b_epilogue():
        o = acc_ref[:, :] + b2_ref[:]
        m = jnp.max(o, axis=1, keepdims=True)
        e = o - m
        lse = jnp.log(jnp.sum(jnp.exp(e), axis=1, keepdims=True))
        o_ref[:, :] = e - lse


def _schedule():
    rows = []
    for i in range(_TA):
        rows.append((i, 0, 0, 1, 1, i, 0, 0, 0))
    for j in range(_TB):
        k0 = j // 2
        for k in range(k0, _K):
            rows.append((j, k, 1, int(k == k0), int(k == _K - 1),
                         _TA - 1, j, k, j))
    return np.asarray(rows, dtype=np.int32).T


@jax.jit
def _run(x, adj, W1, b1, W2, b2):
    N, nfeat = x.shape
    nhid = W1.shape[1]
    nclass = W2.shape[1]

    g = pl.pallas_call(
        _xw_kernel,
        out_shape=jax.ShapeDtypeStruct((N, nhid), jnp.float32),
    )(x, W1)

    info = jnp.asarray(_schedule())
    nsteps = info.shape[1]

    grid_spec = pltpu.PrefetchScalarGridSpec(
        num_scalar_prefetch=1,
        grid=(nsteps,),
        in_specs=[
            pl.BlockSpec((_BA, N), lambda t, info: (info[5, t], 0)),
            pl.BlockSpec((_BB, _W), lambda t, info: (info[6, t], info[7, t])),
            pl.BlockSpec((N, nhid), lambda t, info: (0, 0)),
            pl.BlockSpec((nhid,), lambda t, info: (0,)),
            pl.BlockSpec((nhid, nclass), lambda t, info: (0, 0)),
            pl.BlockSpec((nclass,), lambda t, info: (0,)),
        ],
        out_specs=pl.BlockSpec((_BB, nclass), lambda t, info: (info[8, t], 0)),
        scratch_shapes=[
            pltpu.VMEM((_NPAD, _NF), jnp.float32),
            pltpu.VMEM((_NPAD, _NC), jnp.float32),
            pltpu.VMEM((_NPAD, _NC), jnp.float32),
            pltpu.VMEM((_BB, _NC), jnp.float32),
        ],
    )

    out = pl.pallas_call(
        _main_kernel,
        grid_spec=grid_spec,
        out_shape=jax.ShapeDtypeStruct((N, nclass), jnp.float32),
    )(info, adj, adj, g, b1, W2, b2)

    return out


def kernel(x, adj, W1, b1, W2, b2, epoch, test):
    del epoch, test
    return _run(x, adj, W1, b1, W2, b2)


# speedup vs baseline: 2.1773x; 1.0214x over previous
import jax
import jax.numpy as jnp
import numpy as np
from jax.experimental import pallas as pl
from jax.experimental.pallas import tpu as pltpu

_N = 10000
_BA = 256
_TA = (_N + _BA - 1) // _BA
_BB = 1024
_TB = (_N + _BB - 1) // _BB
_W = 2048
_K = (_N + _W - 1) // _W
_EDGE = _N - (_K - 1) * _W
_CHUNK = _W // _BA
_NH = 64
_NC = 16
_NF = _NH + _NC
_S0 = _NF
_S1 = _NF + _NC
_SW = _NF + 2 * _NC
_NPAD = _TA * _BA


def _main_kernel(info_ref, adja_ref, adjb_ref, x_ref, w1_ref, b1_ref, w2_ref,
                 b2_ref, o_ref, gp_ref, acc_ref):
    t = pl.program_id(0)
    i = info_ref[0, t]
    k = info_ref[1, t]
    phase = info_ref[2, t]
    first = info_ref[3, t]
    last = info_ref[4, t]

    @pl.when(t == 0)
    def _init():
        gp_ref[:, :] = jnp.zeros((_NPAD, _SW), jnp.float32)
        gp_ref[0:_N, 0:_NH] = jnp.dot(x_ref[:, :], w1_ref[:, :],
                                      preferred_element_type=jnp.float32)

    @pl.when(phase == 0)
    def _a_step():
        @pl.when((i % _CHUNK == 0) & (i > 0))
        def _expose():
            c = i // _CHUNK - 1
            gp_ref[pl.ds(c * _W, _W), _NH:_NF] = (
                gp_ref[pl.ds(c * _W, _W), _S0:_S1])

        hp = jnp.dot(adja_ref[:, :], gp_ref[0:_N, 0:_NF],
                     preferred_element_type=jnp.float32)
        h = jnp.maximum(hp[:, 0:_NH] + b1_ref[:], 0.0)
        pb = jnp.dot(h, w2_ref[:, :], preferred_element_type=jnp.float32)
        gp_ref[pl.ds(i * _BA, _BA), _S0:_S1] = pb
        gp_ref[pl.ds(i * _BA, _BA), _S1:_SW] = hp[:, _NH:_NF]

        @pl.when(i == _TA - 1)
        def _finish_p():
            gp_ref[_N:_NPAD, _S0:_S1] = jnp.zeros((_NPAD - _N, _NC),
                                                  jnp.float32)
            gp_ref[pl.ds((_K - 1) * _W, _W), _NH:_NF] = (
                gp_ref[pl.ds((_K - 1) * _W, _W), _S0:_S1])

    @pl.when((phase == 1) & (first == 1))
    def _b_load():
        acc_ref[:, :] = gp_ref[pl.ds(i * _BB, _BB), _S1:_SW]

    @pl.when((phase == 1) & (k != _K - 1))
    def _b_main():
        acc_ref[:, :] += jnp.dot(adjb_ref[:, :],
                                 gp_ref[pl.ds(k * _W, _W), _NH:_NF],
                                 preferred_element_type=jnp.float32)

    @pl.when((phase == 1) & (k == _K - 1))
    def _b_edge():
        acc_ref[:, :] += jnp.dot(adjb_ref[:, 0:_EDGE],
                                 gp_ref[pl.ds(k * _W, _EDGE), _NH:_NF],
                                 preferred_element_type=jnp.float32)

    @pl.when((phase == 1) & (last == 1))
    def _b_epilogue():
        o = acc_ref[:, :] + b2_ref[:]
        m = jnp.max(o, axis=1, keepdims=True)
        e = o - m
        lse = jnp.log(jnp.sum(jnp.exp(e), axis=1, keepdims=True))
        o_ref[:, :] = e - lse


def _schedule():
    rows = []
    for i in range(_TA):
        rows.append((i, 0, 0, 1, 1, i, 0, 0, 0))
    for j in range(_TB):
        k0 = j // 2
        for k in range(k0, _K):
            rows.append((j, k, 1, int(k == k0), int(k == _K - 1),
                         _TA - 1, j, k, j))
    return np.asarray(rows, dtype=np.int32).T


@jax.jit
def _run(x, adj, W1, b1, W2, b2):
    N, nfeat = x.shape
    nhid = W1.shape[1]
    nclass = W2.shape[1]

    info = jnp.asarray(_schedule())
    nsteps = info.shape[1]

    grid_spec = pltpu.PrefetchScalarGridSpec(
        num_scalar_prefetch=1,
        grid=(nsteps,),
        in_specs=[
            pl.BlockSpec((_BA, N), lambda t, info: (info[5, t], 0)),
            pl.BlockSpec((_BB, _W), lambda t, info: (info[6, t], info[7, t])),
            pl.BlockSpec((N, nfeat), lambda t, info: (0, 0)),
            pl.BlockSpec((nfeat, nhid), lambda t, info: (0, 0)),
            pl.BlockSpec((nhid,), lambda t, info: (0,)),
            pl.BlockSpec((nhid, nclass), lambda t, info: (0, 0)),
            pl.BlockSpec((nclass,), lambda t, info: (0,)),
        ],
        out_specs=pl.BlockSpec((_BB, nclass), lambda t, info: (info[8, t], 0)),
        scratch_shapes=[
            pltpu.VMEM((_NPAD, _SW), jnp.float32),
            pltpu.VMEM((_BB, _NC), jnp.float32),
        ],
    )

    out = pl.pallas_call(
        _main_kernel,
        grid_spec=grid_spec,
        out_shape=jax.ShapeDtypeStruct((N, nclass), jnp.float32),
    )(info, adj, adj, x, W1, b1, W2, b2)

    return out


def kernel(x, adj, W1, b1, W2, b2, epoch, test):
    del epoch, test
    return _run(x, adj, W1, b1, W2, b2)
